# Initial kernel scaffold; baseline (speedup 1.0000x reference)
import jax
import jax.numpy as jnp
from jax.experimental import pallas as pl


def kernel(x, edge_index, W1, b1, W2, b2):
    raise NotImplementedError("write your pallas kernel here")


# speedup vs baseline: 11.7770x; 11.7770x over previous
import functools

import jax
import jax.numpy as jnp
from jax import lax
from jax.experimental import pallas as pl
from jax.experimental.pallas import tpu as pltpu
from jax.experimental.pallas import tpu_sc as plsc

f32 = jnp.float32
i32 = jnp.int32

N_NODES = 10000
NPAD = 10240
NFEAT = 256
FH = 128
HIDDEN = 512
NCLASS = 40
CPAD = 64
N_EDGES = 160000
E_ALL = N_EDGES + N_NODES
CHUNK = 128
CH_C = 84
CH_E = 42
EPAD = 32 * CH_E * CHUNK
DUMMY = N_NODES
RPT = NPAD // 16
MB = 512

_mesh = plsc.VectorSubcoreMesh(
    core_axis_name="c", subcore_axis_name="s", num_cores=2, num_subcores=16
)


def _deg_body(dst_hbm, degp_hbm, dst_v, deg_v):
    c = lax.axis_index("c")
    s = lax.axis_index("s")
    wid = c * 16 + s
    pltpu.sync_copy(dst_hbm.at[wid], dst_v)
    zeros16 = jnp.zeros((16,), f32)

    def zero_body(i, _):
        deg_v[pl.ds(i * 16, 16)] = zeros16
        return ()

    lax.fori_loop(0, NPAD // 16, zero_body, (), unroll=8)
    ones16 = jnp.ones((16,), f32)

    def cnt_body(i, _):
        idx = dst_v[pl.ds(i * 16, 16)]
        plsc.addupdate_scatter(deg_v, [idx], ones16)
        return ()

    lax.fori_loop(0, (CH_E * CHUNK) // 16, cnt_body, (), unroll=8)
    pltpu.sync_copy(deg_v, degp_hbm.at[wid])


_deg_kernel = pl.kernel(
    _deg_body,
    out_type=jax.ShapeDtypeStruct((32, NPAD), f32),
    mesh=_mesh,
    compiler_params=pltpu.CompilerParams(needs_layout_passes=False),
    scratch_types=[
        pltpu.VMEM((CH_E * CHUNK,), i32),
        pltpu.VMEM((NPAD,), f32),
    ],
)


def _prep_body(degp_ref, x_ref, xs0_ref, xs1_ref, dis_ref):
    deg = jnp.sum(degp_ref[...], axis=0)
    dis = lax.rsqrt(jnp.maximum(deg, 1.0))
    xs = x_ref[...] * dis[:, None]
    xs0_ref[...] = xs[:, :FH]
    xs1_ref[...] = xs[:, FH:]
    dis_ref[...] = jnp.broadcast_to(dis[:, None], dis_ref.shape)


_prep = pl.pallas_call(
    _prep_body,
    grid=(NPAD // MB,),
    in_specs=[
        pl.BlockSpec((32, MB), lambda m: (0, m)),
        pl.BlockSpec((MB, NFEAT), lambda m: (m, 0)),
    ],
    out_specs=[
        pl.BlockSpec((MB, FH), lambda m: (m, 0)),
        pl.BlockSpec((MB, FH), lambda m: (m, 0)),
        pl.BlockSpec((MB, FH), lambda m: (m, 0)),
    ],
    out_shape=[
        jax.ShapeDtypeStruct((NPAD, FH), f32),
        jax.ShapeDtypeStruct((NPAD, FH), f32),
        jax.ShapeDtypeStruct((NPAD, FH), f32),
    ],
)


def _agg1_body(xs0_hbm, xs1_hbm, src_hbm, dst_hbm, zero_hbm, out_hbm,
               src_v, dst_v, rows_v, acc_sh, sem):
    c = lax.axis_index("c")
    s = lax.axis_index("s")
    pltpu.sync_copy(src_hbm.at[s], src_v)
    pltpu.sync_copy(dst_hbm.at[s], dst_v)
    pltpu.sync_copy(zero_hbm, rows_v)
    base = s * RPT
    for k in range(RPT // CHUNK):
        pltpu.sync_copy(rows_v, acc_sh.at[pl.ds(base + k * CHUNK, CHUNK)])
    plsc.subcore_barrier()

    def edge_loop(tab_hbm):
        def body(j, _):
            pltpu.async_copy(tab_hbm.at[src_v.at[j]], rows_v, sem).wait()
            pltpu.sync_copy(rows_v, acc_sh.at[dst_v.at[j]], add=True)
            return ()

        lax.fori_loop(0, CH_C, body, ())

    @pl.when(c == 0)
    def _():
        edge_loop(xs0_hbm)

    @pl.when(c == 1)
    def _():
        edge_loop(xs1_hbm)

    plsc.subcore_barrier()
    out_base = c * NPAD + base
    for k in range(RPT // CHUNK):
        pltpu.sync_copy(acc_sh.at[pl.ds(base + k * CHUNK, CHUNK)], rows_v)
        pltpu.sync_copy(rows_v, out_hbm.at[pl.ds(out_base + k * CHUNK, CHUNK)])


_agg1_kernel = pl.kernel(
    _agg1_body,
    out_type=jax.ShapeDtypeStruct((2 * NPAD, FH), f32),
    mesh=_mesh,
    scratch_types=[
        pltpu.VMEM((CH_C, CHUNK), i32),
        pltpu.VMEM((CH_C, CHUNK), i32),
        pltpu.VMEM((CHUNK, FH), f32),
        pltpu.VMEM_SHARED((NPAD, FH), f32),
        pltpu.SemaphoreType.DMA,
    ],
)


def _dense_body(a0_ref, a1_ref, dis_ref, w1_ref, b1_ref, w2_ref, ys_ref):
    d = dis_ref[...]
    a0 = a0_ref[...] * d
    a1 = a1_ref[...] * d
    w1 = w1_ref[...]
    h = (
        jnp.dot(a0, w1[:FH], preferred_element_type=f32,
                precision=lax.Precision.HIGHEST)
        + jnp.dot(a1, w1[FH:], preferred_element_type=f32,
                  precision=lax.Precision.HIGHEST)
    )
    h = jnp.maximum(h + b1_ref[...], 0.0)
    y = jnp.dot(h, w2_ref[...], preferred_element_type=f32,
                precision=lax.Precision.HIGHEST)
    ys_ref[...] = y * d[:, :CPAD]


_dense = pl.pallas_call(
    _dense_body,
    grid=(NPAD // MB,),
    in_specs=[
        pl.BlockSpec((MB, FH), lambda m: (m, 0)),
        pl.BlockSpec((MB, FH), lambda m: (m + NPAD // MB, 0)),
        pl.BlockSpec((MB, FH), lambda m: (m, 0)),
        pl.BlockSpec((NFEAT, HIDDEN), lambda m: (0, 0)),
        pl.BlockSpec((1, HIDDEN), lambda m: (0, 0)),
        pl.BlockSpec((HIDDEN, CPAD), lambda m: (0, 0)),
    ],
    out_specs=pl.BlockSpec((MB, CPAD), lambda m: (m, 0)),
    out_shape=jax.ShapeDtypeStruct((NPAD, CPAD), f32),
)


def _agg2_body(ys_hbm, src_hbm, dst_hbm, zero_hbm, out_hbm,
               src_v, dst_v, rows_v, acc_sh, sem):
    c = lax.axis_index("c")
    s = lax.axis_index("s")
    wid = c * 16 + s
    pltpu.sync_copy(src_hbm.at[wid], src_v)
    pltpu.sync_copy(dst_hbm.at[wid], dst_v)
    pltpu.sync_copy(zero_hbm, rows_v)
    base = s * RPT
    for k in range(RPT // CHUNK):
        pltpu.sync_copy(rows_v, acc_sh.at[pl.ds(base + k * CHUNK, CHUNK)])
    plsc.subcore_barrier()

    def body(j, _):
        pltpu.async_copy(ys_hbm.at[src_v.at[j]], rows_v, sem).wait()
        pltpu.sync_copy(rows_v, acc_sh.at[dst_v.at[j]], add=True)
        return ()

    lax.fori_loop(0, CH_E, body, ())
    plsc.subcore_barrier()
    out_base = c * NPAD + base
    for k in range(RPT // CHUNK):
        pltpu.sync_copy(acc_sh.at[pl.ds(base + k * CHUNK, CHUNK)], rows_v)
        pltpu.sync_copy(rows_v, out_hbm.at[pl.ds(out_base + k * CHUNK, CHUNK)])


_agg2_kernel = pl.kernel(
    _agg2_body,
    out_type=jax.ShapeDtypeStruct((2 * NPAD, CPAD), f32),
    mesh=_mesh,
    compiler_params=pltpu.CompilerParams(use_tc_tiling_on_sc=False),
    scratch_types=[
        pltpu.VMEM((CH_E, CHUNK), i32),
        pltpu.VMEM((CH_E, CHUNK), i32),
        pltpu.VMEM((CHUNK, CPAD), f32),
        pltpu.VMEM_SHARED((NPAD, CPAD), f32),
        pltpu.SemaphoreType.DMA,
    ],
)


def _out_body(a0_ref, a1_ref, dis_ref, b2_ref, o_ref):
    z = (a0_ref[...] + a1_ref[...]) * dis_ref[...][:, :CPAD] + b2_ref[...]
    col = lax.broadcasted_iota(i32, z.shape, 1)
    z = jnp.where(col < NCLASS, z, -1e30)
    m = jnp.max(z, axis=1, keepdims=True)
    e = jnp.exp(z - m)
    ssum = jnp.sum(e, axis=1, keepdims=True)
    o_ref[...] = z - m - jnp.log(ssum)


_outk = pl.pallas_call(
    _out_body,
    grid=(NPAD // MB,),
    in_specs=[
        pl.BlockSpec((MB, CPAD), lambda m: (m, 0)),
        pl.BlockSpec((MB, CPAD), lambda m: (m + NPAD // MB, 0)),
        pl.BlockSpec((MB, FH), lambda m: (m, 0)),
        pl.BlockSpec((1, CPAD), lambda m: (0, 0)),
    ],
    out_specs=pl.BlockSpec((MB, CPAD), lambda m: (m, 0)),
    out_shape=jax.ShapeDtypeStruct((NPAD, CPAD), f32),
)


def kernel(x, edge_index, W1, b1, W2, b2):
    src = edge_index[0].astype(i32)
    dst = edge_index[1].astype(i32)
    loop = jnp.arange(N_NODES, dtype=i32)
    n_pad_edges = EPAD - E_ALL
    src_pad = jnp.concatenate([src, loop, jnp.zeros((n_pad_edges,), i32)])
    dst_pad = jnp.concatenate([dst, loop, jnp.full((n_pad_edges,), DUMMY, i32)])
    srcC = src_pad.reshape(16, CH_C, CHUNK)
    dstC = dst_pad.reshape(16, CH_C, CHUNK)
    srcE = src_pad.reshape(32, CH_E, CHUNK)
    dstE = dst_pad.reshape(32, CH_E, CHUNK)
    dstA = dst_pad.reshape(32, CH_E * CHUNK)
    x_pad = jnp.concatenate([x, jnp.zeros((NPAD - N_NODES, NFEAT), f32)])
    w2p = jnp.pad(W2, ((0, 0), (0, CPAD - NCLASS)))
    b2p = jnp.pad(b2, (0, CPAD - NCLASS)).reshape(1, CPAD)
    b1r = b1.reshape(1, HIDDEN)
    zero128 = jnp.zeros((CHUNK, FH), f32)
    zero64 = jnp.zeros((CHUNK, CPAD), f32)

    degp = _deg_kernel(dstA)
    xs0, xs1, dis2d = _prep(degp, x_pad)
    agg = _agg1_kernel(xs0, xs1, srcC, dstC, zero128)
    ys = _dense(agg, agg, dis2d, W1, b1r, w2p)
    acc2 = _agg2_kernel(ys, srcE, dstE, zero64)
    o = _outk(acc2, acc2, dis2d, b2p)
    return o[:N_NODES, :NCLASS]
